# Initial kernel scaffold; baseline (speedup 1.0000x reference)
import jax
import jax.numpy as jnp
from jax.experimental import pallas as pl


def kernel(x, edge_index, W1, b1, W2, b2):
    raise NotImplementedError("write your pallas kernel here")


# speedup vs baseline: 76.5715x; 76.5715x over previous
import functools

import jax
import jax.numpy as jnp
from jax import lax
from jax.experimental import pallas as pl
from jax.experimental.pallas import tpu as pltpu
from jax.experimental.pallas import tpu_sc as plsc

N = 100000
E = 6400000
NC = 2
NS = 16
NW = NC * NS
BK = 128
NB = 16
NB1 = 8
BATCHES_PER_W = 1568
EPW = BATCHES_PER_W * BK
E_PAD = NW * EPW
N_ACC = 100352
RPT = N_ACC // NS

_MESH = plsc.VectorSubcoreMesh(core_axis_name="c", subcore_axis_name="s")
_SC_PARAMS = pltpu.CompilerParams(use_tc_tiling_on_sc=False)


def _zero_accum(zrow_v, acc_s, s):
    def zrow_body(i, _):
        zrow_v[i] = jnp.zeros((16,), jnp.float32)
        return 0
    lax.fori_loop(0, zrow_v.shape[0], zrow_body, 0)

    r0 = s * RPT
    chunk = zrow_v.shape[0]

    def zacc_body(k, _):
        pltpu.sync_copy(zrow_v, acc_s.at[pl.ds(r0 + k * chunk, chunk)])
        return 0
    lax.fori_loop(0, RPT // chunk, zacc_body, 0)


def _deg_body(dst_hbm, out_hbm, dst_v, ones_v, zflat_v, acc_s, sem_s):
    c = lax.axis_index("c")
    s = lax.axis_index("s")
    wid = c * NS + s
    for i in range(8):
        ones_v[pl.ds(i * 16, 16)] = jnp.ones((16,), jnp.float32)
    r0 = s * RPT

    def zrow_body(i, _):
        zflat_v[pl.ds(i * 16, 16)] = jnp.zeros((16,), jnp.float32)
        return 0
    lax.fori_loop(0, zflat_v.shape[0] // 16, zrow_body, 0)

    def zacc_body(k, _):
        pltpu.sync_copy(zflat_v, acc_s.at[pl.ds(r0 + k * 2048, 2048)])
        return 0
    lax.fori_loop(0, RPT // 2048, zacc_body, 0)
    pltpu.sync_copy(zflat_v.at[pl.ds(0, 128)],
                    acc_s.at[pl.ds(r0 + 6144, 128)])
    plsc.subcore_barrier()

    row0 = wid * BATCHES_PER_W

    def super_body(sc, _):
        rb = row0 + sc * NB
        pltpu.sync_copy(dst_hbm.at[pl.ds(rb, NB)], dst_v)
        for j in range(NB):
            pltpu.async_copy(ones_v, acc_s.at[dst_v.at[j]], sem_s, add=True)
        for j in range(NB):
            pltpu.make_async_copy(ones_v, acc_s.at[dst_v.at[j]], sem_s).wait()
        return 0
    lax.fori_loop(0, BATCHES_PER_W // NB, super_body, 0)
    plsc.subcore_barrier()
    pltpu.sync_copy(acc_s.at[pl.ds(r0, RPT)],
                    out_hbm.at[pl.ds(c * N_ACC + r0, RPT)])


def _spmm_body(g_hbm, src_hbm, dst_hbm, out_hbm,
               src_v, dst_v, rows_v, zrow_v, acc_s, sem_g, sem_s):
    c = lax.axis_index("c")
    s = lax.axis_index("s")
    wid = c * NS + s
    r0 = s * RPT
    _zero_accum(zrow_v, acc_s, s)
    plsc.subcore_barrier()

    row0 = wid * BATCHES_PER_W
    nb = src_v.shape[0]

    def super_body(sc, _):
        rb = row0 + sc * nb
        pltpu.sync_copy(src_hbm.at[pl.ds(rb, nb)], src_v)
        pltpu.sync_copy(dst_hbm.at[pl.ds(rb, nb)], dst_v)
        for j in range(nb):
            pltpu.async_copy(g_hbm.at[src_v.at[j]], rows_v.at[j], sem_g)
        for j in range(nb):
            pltpu.make_async_copy(g_hbm.at[src_v.at[j]], rows_v.at[j],
                                  sem_g).wait()
        for j in range(nb):
            pltpu.async_copy(rows_v.at[j], acc_s.at[dst_v.at[j]], sem_s,
                             add=True)
        for j in range(nb):
            pltpu.make_async_copy(rows_v.at[j], acc_s.at[dst_v.at[j]],
                                  sem_s).wait()
        return 0
    lax.fori_loop(0, BATCHES_PER_W // nb, super_body, 0)
    plsc.subcore_barrier()
    pltpu.sync_copy(acc_s.at[pl.ds(r0, RPT)],
                    out_hbm.at[pl.ds(c * N_ACC + r0, RPT)])


@functools.partial(
    pl.kernel,
    out_type=jax.ShapeDtypeStruct((NC * N_ACC,), jnp.float32),
    mesh=_MESH,
    compiler_params=_SC_PARAMS,
    scratch_types=[
        pltpu.VMEM((NB, BK), jnp.int32),
        pltpu.VMEM((128,), jnp.float32),
        pltpu.VMEM((2048,), jnp.float32),
        pltpu.VMEM_SHARED((N_ACC,), jnp.float32),
        pltpu.SemaphoreType.DMA,
    ],
)
def _deg_kernel(dst_hbm, out_hbm, dst_v, ones_v, zflat_v, acc_s, sem_s):
    _deg_body(dst_hbm, out_hbm, dst_v, ones_v, zflat_v, acc_s, sem_s)


@functools.partial(
    pl.kernel,
    out_type=jax.ShapeDtypeStruct((NC * N_ACC, 16), jnp.float32),
    mesh=_MESH,
    compiler_params=_SC_PARAMS,
    scratch_types=[
        pltpu.VMEM((NB1, BK), jnp.int32),
        pltpu.VMEM((NB1, BK), jnp.int32),
        pltpu.VMEM((NB1, BK, 16), jnp.float32),
        pltpu.VMEM((64, 16), jnp.float32),
        pltpu.VMEM_SHARED((N_ACC, 16), jnp.float32),
        pltpu.SemaphoreType.DMA,
        pltpu.SemaphoreType.DMA,
    ],
)
def _spmm16_kernel(g_hbm, src_hbm, dst_hbm, out_hbm,
                   src_v, dst_v, rows_v, zrow_v, acc_s, sem_g, sem_s):
    _spmm_body(g_hbm, src_hbm, dst_hbm, out_hbm,
               src_v, dst_v, rows_v, zrow_v, acc_s, sem_g, sem_s)


BN = 1024


def _dense_a_body(degp_ref, x_ref, w1_ref, g1_ref, dinv_ref):
    deg = degp_ref[0, :] + degp_ref[1, :] + 1.0
    dinv = lax.rsqrt(deg)
    h = jnp.dot(x_ref[...], w1_ref[...], preferred_element_type=jnp.float32)
    g1_ref[...] = h * dinv[:, None]
    dinv_ref[...] = dinv


def _dense_a(degp, x_pad, W1):
    return pl.pallas_call(
        _dense_a_body,
        grid=(N_ACC // BN,),
        in_specs=[
            pl.BlockSpec((NC, BN), lambda i: (0, i)),
            pl.BlockSpec((BN, 3), lambda i: (i, 0)),
            pl.BlockSpec((3, 16), lambda i: (0, 0)),
        ],
        out_specs=[
            pl.BlockSpec((BN, 16), lambda i: (i, 0)),
            pl.BlockSpec((BN,), lambda i: (i,)),
        ],
        out_shape=[
            jax.ShapeDtypeStruct((N_ACC, 16), jnp.float32),
            jax.ShapeDtypeStruct((N_ACC,), jnp.float32),
        ],
    )(degp, x_pad, W1)


def _dense_b_body(acc_ref, g1_ref, dinv_ref, b1_ref, u_ref):
    s1 = acc_ref[0] + acc_ref[1] + g1_ref[...]
    dinv = dinv_ref[...]
    out1 = s1 * dinv[:, None] + b1_ref[...][None, :]
    u_ref[...] = jnp.maximum(out1, 0.0) * dinv[:, None]


def _dense_b(acc1, g1, dinv, b1):
    return pl.pallas_call(
        _dense_b_body,
        grid=(N_ACC // BN,),
        in_specs=[
            pl.BlockSpec((NC, BN, 16), lambda i: (0, i, 0)),
            pl.BlockSpec((BN, 16), lambda i: (i, 0)),
            pl.BlockSpec((BN,), lambda i: (i,)),
            pl.BlockSpec((16,), lambda i: (0,)),
        ],
        out_specs=pl.BlockSpec((BN, 16), lambda i: (i, 0)),
        out_shape=jax.ShapeDtypeStruct((N_ACC, 16), jnp.float32),
    )(acc1, g1, dinv, b1)


def _dense_c_body(acc_ref, u_ref, dinv_ref, w2_ref, b2_ref, out_ref):
    t = acc_ref[0] + acc_ref[1] + u_ref[...]
    o = jnp.dot(t, w2_ref[...], preferred_element_type=jnp.float32)
    o = o * dinv_ref[...][:, None] + b2_ref[...][None, :]
    m = jnp.max(o, axis=1, keepdims=True)
    e = jnp.exp(o - m)
    lse = m + jnp.log(jnp.sum(e, axis=1, keepdims=True))
    out_ref[...] = o - lse


def _dense_c(accm, u, dinv, W2, b2):
    return pl.pallas_call(
        _dense_c_body,
        grid=(N_ACC // BN,),
        in_specs=[
            pl.BlockSpec((NC, BN, 16), lambda i: (0, i, 0)),
            pl.BlockSpec((BN, 16), lambda i: (i, 0)),
            pl.BlockSpec((BN,), lambda i: (i,)),
            pl.BlockSpec((16, 2), lambda i: (0, 0)),
            pl.BlockSpec((2,), lambda i: (0,)),
        ],
        out_specs=pl.BlockSpec((BN, 2), lambda i: (i, 0)),
        out_shape=jax.ShapeDtypeStruct((N_ACC, 2), jnp.float32),
    )(accm, u, dinv, W2, b2)


@jax.jit
def kernel(x, edge_index, W1, b1, W2, b2):
    pad = E_PAD - E
    padv = (jnp.arange(pad, dtype=jnp.int32) % 256) + N
    src2d = jnp.concatenate([edge_index[0], padv]).reshape(E_PAD // BK, BK)
    dst2d = jnp.concatenate([edge_index[1], padv]).reshape(E_PAD // BK, BK)
    x_pad = jnp.concatenate(
        [x, jnp.zeros((N_ACC - N, 3), jnp.float32)], axis=0)

    degp = _deg_kernel(dst2d).reshape(NC, N_ACC)
    g1, dinv = _dense_a(degp, x_pad, W1)
    acc1 = _spmm16_kernel(g1, src2d, dst2d).reshape(NC, N_ACC, 16)
    u = _dense_b(acc1, g1, dinv, b1)
    accm = _spmm16_kernel(u, src2d, dst2d).reshape(NC, N_ACC, 16)
    return _dense_c(accm, u, dinv, W2, b2)[:N]


# speedup vs baseline: 105.1234x; 1.3729x over previous
import functools

import jax
import jax.numpy as jnp
from jax import lax
from jax.experimental import pallas as pl
from jax.experimental.pallas import tpu as pltpu
from jax.experimental.pallas import tpu_sc as plsc

N = 100000
E = 6400000
NC = 2
NS = 16
NW = NC * NS
BK = 128
NB = 16
NB1 = 8
BATCHES_PER_W = 1568
EPW = BATCHES_PER_W * BK
E_PAD = NW * EPW
N_ACC = 100352
RPT = N_ACC // NS

_MESH = plsc.VectorSubcoreMesh(core_axis_name="c", subcore_axis_name="s")
_SC_PARAMS = pltpu.CompilerParams(use_tc_tiling_on_sc=False)


def _zero_accum(zrow_v, acc_s, s):
    def zrow_body(i, _):
        zrow_v[i] = jnp.zeros((16,), jnp.float32)
        return 0
    lax.fori_loop(0, zrow_v.shape[0], zrow_body, 0)

    r0 = s * RPT
    chunk = zrow_v.shape[0]

    def zacc_body(k, _):
        pltpu.sync_copy(zrow_v, acc_s.at[pl.ds(r0 + k * chunk, chunk)])
        return 0
    lax.fori_loop(0, RPT // chunk, zacc_body, 0)


def _deg_body(dst_hbm, out_hbm, dst_v, ones_v, zflat_v, acc_s, sem_s):
    c = lax.axis_index("c")
    s = lax.axis_index("s")
    wid = c * NS + s
    for i in range(8):
        ones_v[pl.ds(i * 16, 16)] = jnp.ones((16,), jnp.float32)
    r0 = s * RPT

    def zrow_body(i, _):
        zflat_v[pl.ds(i * 16, 16)] = jnp.zeros((16,), jnp.float32)
        return 0
    lax.fori_loop(0, zflat_v.shape[0] // 16, zrow_body, 0)

    def zacc_body(k, _):
        pltpu.sync_copy(zflat_v, acc_s.at[pl.ds(r0 + k * 2048, 2048)])
        return 0
    lax.fori_loop(0, RPT // 2048, zacc_body, 0)
    pltpu.sync_copy(zflat_v.at[pl.ds(0, 128)],
                    acc_s.at[pl.ds(r0 + 6144, 128)])
    plsc.subcore_barrier()

    row0 = wid * BATCHES_PER_W

    def super_body(sc, _):
        rb = row0 + sc * NB
        pltpu.sync_copy(dst_hbm.at[pl.ds(rb, NB)], dst_v)
        for j in range(NB):
            pltpu.async_copy(ones_v, acc_s.at[dst_v.at[j]], sem_s, add=True)
        for j in range(NB):
            pltpu.make_async_copy(ones_v, acc_s.at[dst_v.at[j]], sem_s).wait()
        return 0
    lax.fori_loop(0, BATCHES_PER_W // NB, super_body, 0)
    plsc.subcore_barrier()
    pltpu.sync_copy(acc_s.at[pl.ds(r0, RPT)],
                    out_hbm.at[pl.ds(c * N_ACC + r0, RPT)])


def _spmm_body(g_hbm, src_hbm, dst_hbm, out_hbm,
               src_v, dst_v, rows_v, zrow_v, acc_s, sem_i, sem_g, sem_s):
    c = lax.axis_index("c")
    s = lax.axis_index("s")
    wid = c * NS + s
    r0 = s * RPT
    _zero_accum(zrow_v, acc_s, s)
    plsc.subcore_barrier()

    row0 = wid * BATCHES_PER_W
    nb = src_v.shape[1]
    n_super = BATCHES_PER_W // nb

    def idx_copies(sc, buf):
        rb = row0 + sc * nb
        return (pltpu.make_async_copy(src_hbm.at[pl.ds(rb, nb)],
                                      src_v.at[buf], sem_i),
                pltpu.make_async_copy(dst_hbm.at[pl.ds(rb, nb)],
                                      dst_v.at[buf], sem_i))

    pltpu.sync_copy(src_hbm.at[pl.ds(row0, nb)], src_v.at[0])
    pltpu.sync_copy(dst_hbm.at[pl.ds(row0, nb)], dst_v.at[0])
    for cp in idx_copies(1, 1):
        cp.start()

    def super_body(sc, _):
        cur = lax.rem(sc, 2)
        @pl.when(sc > 0)
        def _():
            for cp in idx_copies(sc, cur):
                cp.wait()
        @pl.when(sc + 1 < n_super)
        def _():
            @pl.when(sc > 0)
            def _():
                for cp in idx_copies(sc + 1, 1 - cur):
                    cp.start()
        for j in range(nb):
            pltpu.async_copy(g_hbm.at[src_v.at[cur, j]], rows_v.at[j], sem_g)
        for j in range(nb):
            pltpu.make_async_copy(g_hbm.at[src_v.at[cur, j]], rows_v.at[j],
                                  sem_g).wait()
            pltpu.async_copy(rows_v.at[j], acc_s.at[dst_v.at[cur, j]], sem_s,
                             add=True)
        for j in range(nb):
            pltpu.make_async_copy(rows_v.at[j], acc_s.at[dst_v.at[cur, j]],
                                  sem_s).wait()
        return 0
    lax.fori_loop(0, n_super, super_body, 0)
    plsc.subcore_barrier()
    pltpu.sync_copy(acc_s.at[pl.ds(r0, RPT)],
                    out_hbm.at[pl.ds(c * N_ACC + r0, RPT)])


@functools.partial(
    pl.kernel,
    out_type=jax.ShapeDtypeStruct((NC * N_ACC,), jnp.float32),
    mesh=_MESH,
    compiler_params=_SC_PARAMS,
    scratch_types=[
        pltpu.VMEM((NB, BK), jnp.int32),
        pltpu.VMEM((128,), jnp.float32),
        pltpu.VMEM((2048,), jnp.float32),
        pltpu.VMEM_SHARED((N_ACC,), jnp.float32),
        pltpu.SemaphoreType.DMA,
    ],
)
def _deg_kernel(dst_hbm, out_hbm, dst_v, ones_v, zflat_v, acc_s, sem_s):
    _deg_body(dst_hbm, out_hbm, dst_v, ones_v, zflat_v, acc_s, sem_s)


@functools.partial(
    pl.kernel,
    out_type=jax.ShapeDtypeStruct((NC * N_ACC, 16), jnp.float32),
    mesh=_MESH,
    compiler_params=_SC_PARAMS,
    scratch_types=[
        pltpu.VMEM((2, NB1, BK), jnp.int32),
        pltpu.VMEM((2, NB1, BK), jnp.int32),
        pltpu.VMEM((NB1, BK, 16), jnp.float32),
        pltpu.VMEM((64, 16), jnp.float32),
        pltpu.VMEM_SHARED((N_ACC, 16), jnp.float32),
        pltpu.SemaphoreType.DMA,
        pltpu.SemaphoreType.DMA,
        pltpu.SemaphoreType.DMA,
    ],
)
def _spmm16_kernel(g_hbm, src_hbm, dst_hbm, out_hbm,
                   src_v, dst_v, rows_v, zrow_v, acc_s, sem_i, sem_g, sem_s):
    _spmm_body(g_hbm, src_hbm, dst_hbm, out_hbm,
               src_v, dst_v, rows_v, zrow_v, acc_s, sem_i, sem_g, sem_s)


BN = 1024


def _dense_a_body(degp_ref, x_ref, w1_ref, g1_ref, dinv_ref):
    deg = degp_ref[0, :] + degp_ref[1, :] + 1.0
    dinv = lax.rsqrt(deg)
    h = jnp.dot(x_ref[...], w1_ref[...], preferred_element_type=jnp.float32)
    g1_ref[...] = h * dinv[:, None]
    dinv_ref[...] = dinv


def _dense_a(degp, x_pad, W1):
    return pl.pallas_call(
        _dense_a_body,
        grid=(N_ACC // BN,),
        in_specs=[
            pl.BlockSpec((NC, BN), lambda i: (0, i)),
            pl.BlockSpec((BN, 3), lambda i: (i, 0)),
            pl.BlockSpec((3, 16), lambda i: (0, 0)),
        ],
        out_specs=[
            pl.BlockSpec((BN, 16), lambda i: (i, 0)),
            pl.BlockSpec((BN,), lambda i: (i,)),
        ],
        out_shape=[
            jax.ShapeDtypeStruct((N_ACC, 16), jnp.float32),
            jax.ShapeDtypeStruct((N_ACC,), jnp.float32),
        ],
    )(degp, x_pad, W1)


def _dense_b_body(acc_ref, g1_ref, dinv_ref, b1_ref, u_ref):
    s1 = acc_ref[0] + acc_ref[1] + g1_ref[...]
    dinv = dinv_ref[...]
    out1 = s1 * dinv[:, None] + b1_ref[...][None, :]
    u_ref[...] = jnp.maximum(out1, 0.0) * dinv[:, None]


def _dense_b(acc1, g1, dinv, b1):
    return pl.pallas_call(
        _dense_b_body,
        grid=(N_ACC // BN,),
        in_specs=[
            pl.BlockSpec((NC, BN, 16), lambda i: (0, i, 0)),
            pl.BlockSpec((BN, 16), lambda i: (i, 0)),
            pl.BlockSpec((BN,), lambda i: (i,)),
            pl.BlockSpec((16,), lambda i: (0,)),
        ],
        out_specs=pl.BlockSpec((BN, 16), lambda i: (i, 0)),
        out_shape=jax.ShapeDtypeStruct((N_ACC, 16), jnp.float32),
    )(acc1, g1, dinv, b1)


def _dense_c_body(acc_ref, u_ref, dinv_ref, w2_ref, b2_ref, out_ref):
    t = acc_ref[0] + acc_ref[1] + u_ref[...]
    o = jnp.dot(t, w2_ref[...], preferred_element_type=jnp.float32)
    o = o * dinv_ref[...][:, None] + b2_ref[...][None, :]
    m = jnp.max(o, axis=1, keepdims=True)
    e = jnp.exp(o - m)
    lse = m + jnp.log(jnp.sum(e, axis=1, keepdims=True))
    out_ref[...] = o - lse


def _dense_c(accm, u, dinv, W2, b2):
    return pl.pallas_call(
        _dense_c_body,
        grid=(N_ACC // BN,),
        in_specs=[
            pl.BlockSpec((NC, BN, 16), lambda i: (0, i, 0)),
            pl.BlockSpec((BN, 16), lambda i: (i, 0)),
            pl.BlockSpec((BN,), lambda i: (i,)),
            pl.BlockSpec((16, 2), lambda i: (0, 0)),
            pl.BlockSpec((2,), lambda i: (0,)),
        ],
        out_specs=pl.BlockSpec((BN, 2), lambda i: (i, 0)),
        out_shape=jax.ShapeDtypeStruct((N_ACC, 2), jnp.float32),
    )(accm, u, dinv, W2, b2)


@jax.jit
def kernel(x, edge_index, W1, b1, W2, b2):
    pad = E_PAD - E
    padv = (jnp.arange(pad, dtype=jnp.int32) % 256) + N
    src2d = jnp.concatenate([edge_index[0], padv]).reshape(E_PAD // BK, BK)
    dst2d = jnp.concatenate([edge_index[1], padv]).reshape(E_PAD // BK, BK)
    x_pad = jnp.concatenate(
        [x, jnp.zeros((N_ACC - N, 3), jnp.float32)], axis=0)

    degp = _deg_kernel(dst2d).reshape(NC, N_ACC)
    g1, dinv = _dense_a(degp, x_pad, W1)
    acc1 = _spmm16_kernel(g1, src2d, dst2d).reshape(NC, N_ACC, 16)
    u = _dense_b(acc1, g1, dinv, b1)
    accm = _spmm16_kernel(u, src2d, dst2d).reshape(NC, N_ACC, 16)
    return _dense_c(accm, u, dinv, W2, b2)[:N]


# speedup vs baseline: 108.4755x; 1.0319x over previous
import functools

import jax
import jax.numpy as jnp
from jax import lax
from jax.experimental import pallas as pl
from jax.experimental.pallas import tpu as pltpu
from jax.experimental.pallas import tpu_sc as plsc

N = 100000
E = 6400000
NC = 2
NS = 16
NW = NC * NS
BK = 128
NB = 16
NB1 = 8
BATCHES_PER_W = 1568
EPW = BATCHES_PER_W * BK
E_PAD = NW * EPW
N_ACC = 100352
RPT = N_ACC // NS

_MESH = plsc.VectorSubcoreMesh(core_axis_name="c", subcore_axis_name="s")
_SC_PARAMS = pltpu.CompilerParams(use_tc_tiling_on_sc=False)


def _zero_accum(zrow_v, acc_s, s):
    def zrow_body(i, _):
        zrow_v[i] = jnp.zeros((16,), jnp.float32)
        return 0
    lax.fori_loop(0, zrow_v.shape[0], zrow_body, 0)

    r0 = s * RPT
    chunk = zrow_v.shape[0]

    def zacc_body(k, _):
        pltpu.sync_copy(zrow_v, acc_s.at[pl.ds(r0 + k * chunk, chunk)])
        return 0
    lax.fori_loop(0, RPT // chunk, zacc_body, 0)


def _deg_body(dst_hbm, out_hbm, dst_v, ones_v, zflat_v, acc_s, sem_i, sem_s):
    c = lax.axis_index("c")
    s = lax.axis_index("s")
    wid = c * NS + s
    for i in range(8):
        ones_v[pl.ds(i * 16, 16)] = jnp.ones((16,), jnp.float32)
    r0 = s * RPT

    def zrow_body(i, _):
        zflat_v[pl.ds(i * 16, 16)] = jnp.zeros((16,), jnp.float32)
        return 0
    lax.fori_loop(0, zflat_v.shape[0] // 16, zrow_body, 0)

    def zacc_body(k, _):
        pltpu.sync_copy(zflat_v, acc_s.at[pl.ds(r0 + k * 2048, 2048)])
        return 0
    lax.fori_loop(0, RPT // 2048, zacc_body, 0)
    pltpu.sync_copy(zflat_v.at[pl.ds(0, 128)],
                    acc_s.at[pl.ds(r0 + 6144, 128)])
    plsc.subcore_barrier()

    row0 = wid * BATCHES_PER_W
    n_super = BATCHES_PER_W // NB

    def idx_copy(sc, buf):
        rb = row0 + sc * NB
        return pltpu.make_async_copy(dst_hbm.at[pl.ds(rb, NB)],
                                     dst_v.at[buf], sem_i)

    pltpu.sync_copy(dst_hbm.at[pl.ds(row0, NB)], dst_v.at[0])
    idx_copy(1, 1).start()

    def super_body(sc, _):
        cur = lax.rem(sc, 2)

        @pl.when(sc > 0)
        def _():
            idx_copy(sc, cur).wait()

        @pl.when((sc + 1 < n_super) & (sc > 0))
        def _():
            idx_copy(sc + 1, 1 - cur).start()

        for j in range(NB):
            pltpu.async_copy(ones_v, acc_s.at[dst_v.at[cur, j]], sem_s,
                             add=True)
        for j in range(NB):
            pltpu.make_async_copy(ones_v, acc_s.at[dst_v.at[cur, j]],
                                  sem_s).wait()
        return 0
    lax.fori_loop(0, n_super, super_body, 0)
    plsc.subcore_barrier()
    pltpu.sync_copy(acc_s.at[pl.ds(r0, RPT)],
                    out_hbm.at[pl.ds(c * N_ACC + r0, RPT)])


def _spmm_body(g_hbm, src_hbm, dst_hbm, out_hbm,
               src_v, dst_v, rows_v, zrow_v, acc_s, sem_i, sem_g, sem_s):
    c = lax.axis_index("c")
    s = lax.axis_index("s")
    wid = c * NS + s
    r0 = s * RPT
    _zero_accum(zrow_v, acc_s, s)
    plsc.subcore_barrier()

    row0 = wid * BATCHES_PER_W
    nb = src_v.shape[1]
    n_super = BATCHES_PER_W // nb

    def idx_copies(sc, buf):
        rb = row0 + sc * nb
        return (pltpu.make_async_copy(src_hbm.at[pl.ds(rb, nb)],
                                      src_v.at[buf], sem_i),
                pltpu.make_async_copy(dst_hbm.at[pl.ds(rb, nb)],
                                      dst_v.at[buf], sem_i))

    pltpu.sync_copy(src_hbm.at[pl.ds(row0, nb)], src_v.at[0])
    pltpu.sync_copy(dst_hbm.at[pl.ds(row0, nb)], dst_v.at[0])
    for cp in idx_copies(1, 1):
        cp.start()

    def super_body(sc, _):
        cur = lax.rem(sc, 2)
        @pl.when(sc > 0)
        def _():
            for cp in idx_copies(sc, cur):
                cp.wait()
        @pl.when(sc + 1 < n_super)
        def _():
            @pl.when(sc > 0)
            def _():
                for cp in idx_copies(sc + 1, 1 - cur):
                    cp.start()
        for j in range(nb):
            pltpu.async_copy(g_hbm.at[src_v.at[cur, j]], rows_v.at[j], sem_g)
        for j in range(nb):
            pltpu.make_async_copy(g_hbm.at[src_v.at[cur, j]], rows_v.at[j],
                                  sem_g).wait()
            pltpu.async_copy(rows_v.at[j], acc_s.at[dst_v.at[cur, j]], sem_s,
                             add=True)
        for j in range(nb):
            pltpu.make_async_copy(rows_v.at[j], acc_s.at[dst_v.at[cur, j]],
                                  sem_s).wait()
        return 0
    lax.fori_loop(0, n_super, super_body, 0)
    plsc.subcore_barrier()
    pltpu.sync_copy(acc_s.at[pl.ds(r0, RPT)],
                    out_hbm.at[pl.ds(c * N_ACC + r0, RPT)])


@functools.partial(
    pl.kernel,
    out_type=jax.ShapeDtypeStruct((NC * N_ACC,), jnp.float32),
    mesh=_MESH,
    compiler_params=_SC_PARAMS,
    scratch_types=[
        pltpu.VMEM((2, NB, BK), jnp.int32),
        pltpu.VMEM((128,), jnp.float32),
        pltpu.VMEM((2048,), jnp.float32),
        pltpu.VMEM_SHARED((N_ACC,), jnp.float32),
        pltpu.SemaphoreType.DMA,
        pltpu.SemaphoreType.DMA,
    ],
)
def _deg_kernel(dst_hbm, out_hbm, dst_v, ones_v, zflat_v, acc_s, sem_i, sem_s):
    _deg_body(dst_hbm, out_hbm, dst_v, ones_v, zflat_v, acc_s, sem_i, sem_s)


@functools.partial(
    pl.kernel,
    out_type=jax.ShapeDtypeStruct((NC * N_ACC, 16), jnp.float32),
    mesh=_MESH,
    compiler_params=_SC_PARAMS,
    scratch_types=[
        pltpu.VMEM((2, NB1, BK), jnp.int32),
        pltpu.VMEM((2, NB1, BK), jnp.int32),
        pltpu.VMEM((NB1, BK, 16), jnp.float32),
        pltpu.VMEM((64, 16), jnp.float32),
        pltpu.VMEM_SHARED((N_ACC, 16), jnp.float32),
        pltpu.SemaphoreType.DMA,
        pltpu.SemaphoreType.DMA,
        pltpu.SemaphoreType.DMA,
    ],
)
def _spmm16_kernel(g_hbm, src_hbm, dst_hbm, out_hbm,
                   src_v, dst_v, rows_v, zrow_v, acc_s, sem_i, sem_g, sem_s):
    _spmm_body(g_hbm, src_hbm, dst_hbm, out_hbm,
               src_v, dst_v, rows_v, zrow_v, acc_s, sem_i, sem_g, sem_s)


BN = 1024


def _dense_a_body(degp_ref, x_ref, w1_ref, g1_ref, dinv_ref):
    deg = degp_ref[0, :] + degp_ref[1, :] + 1.0
    dinv = lax.rsqrt(deg)
    h = jnp.dot(x_ref[...], w1_ref[...], preferred_element_type=jnp.float32)
    g1_ref[...] = h * dinv[:, None]
    dinv_ref[...] = dinv


def _dense_a(degp, x_pad, W1):
    return pl.pallas_call(
        _dense_a_body,
        grid=(N_ACC // BN,),
        in_specs=[
            pl.BlockSpec((NC, BN), lambda i: (0, i)),
            pl.BlockSpec((BN, 3), lambda i: (i, 0)),
            pl.BlockSpec((3, 16), lambda i: (0, 0)),
        ],
        out_specs=[
            pl.BlockSpec((BN, 16), lambda i: (i, 0)),
            pl.BlockSpec((BN,), lambda i: (i,)),
        ],
        out_shape=[
            jax.ShapeDtypeStruct((N_ACC, 16), jnp.float32),
            jax.ShapeDtypeStruct((N_ACC,), jnp.float32),
        ],
    )(degp, x_pad, W1)


def _dense_b_body(acc_ref, g1_ref, dinv_ref, b1_ref, u_ref):
    s1 = acc_ref[0] + acc_ref[1] + g1_ref[...]
    dinv = dinv_ref[...]
    out1 = s1 * dinv[:, None] + b1_ref[...][None, :]
    u_ref[...] = jnp.maximum(out1, 0.0) * dinv[:, None]


def _dense_b(acc1, g1, dinv, b1):
    return pl.pallas_call(
        _dense_b_body,
        grid=(N_ACC // BN,),
        in_specs=[
            pl.BlockSpec((NC, BN, 16), lambda i: (0, i, 0)),
            pl.BlockSpec((BN, 16), lambda i: (i, 0)),
            pl.BlockSpec((BN,), lambda i: (i,)),
            pl.BlockSpec((16,), lambda i: (0,)),
        ],
        out_specs=pl.BlockSpec((BN, 16), lambda i: (i, 0)),
        out_shape=jax.ShapeDtypeStruct((N_ACC, 16), jnp.float32),
    )(acc1, g1, dinv, b1)


def _dense_c_body(acc_ref, u_ref, dinv_ref, w2_ref, b2_ref, out_ref):
    t = acc_ref[0] + acc_ref[1] + u_ref[...]
    o = jnp.dot(t, w2_ref[...], preferred_element_type=jnp.float32)
    o = o * dinv_ref[...][:, None] + b2_ref[...][None, :]
    m = jnp.max(o, axis=1, keepdims=True)
    e = jnp.exp(o - m)
    lse = m + jnp.log(jnp.sum(e, axis=1, keepdims=True))
    out_ref[...] = o - lse


def _dense_c(accm, u, dinv, W2, b2):
    return pl.pallas_call(
        _dense_c_body,
        grid=(N_ACC // BN,),
        in_specs=[
            pl.BlockSpec((NC, BN, 16), lambda i: (0, i, 0)),
            pl.BlockSpec((BN, 16), lambda i: (i, 0)),
            pl.BlockSpec((BN,), lambda i: (i,)),
            pl.BlockSpec((16, 2), lambda i: (0, 0)),
            pl.BlockSpec((2,), lambda i: (0,)),
        ],
        out_specs=pl.BlockSpec((BN, 2), lambda i: (i, 0)),
        out_shape=jax.ShapeDtypeStruct((N_ACC, 2), jnp.float32),
    )(accm, u, dinv, W2, b2)


@jax.jit
def kernel(x, edge_index, W1, b1, W2, b2):
    pad = E_PAD - E
    padv = (jnp.arange(pad, dtype=jnp.int32) % 256) + N
    src2d = jnp.concatenate([edge_index[0], padv]).reshape(E_PAD // BK, BK)
    dst2d = jnp.concatenate([edge_index[1], padv]).reshape(E_PAD // BK, BK)
    x_pad = jnp.concatenate(
        [x, jnp.zeros((N_ACC - N, 3), jnp.float32)], axis=0)

    degp = _deg_kernel(dst2d).reshape(NC, N_ACC)
    g1, dinv = _dense_a(degp, x_pad, W1)
    acc1 = _spmm16_kernel(g1, src2d, dst2d).reshape(NC, N_ACC, 16)
    u = _dense_b(acc1, g1, dinv, b1)
    accm = _spmm16_kernel(u, src2d, dst2d).reshape(NC, N_ACC, 16)
    return _dense_c(accm, u, dinv, W2, b2)[:N]
